# baseline (speedup 1.0000x reference)
import functools

import jax
import jax.numpy as jnp
from jax import lax
from jax.experimental import pallas as pl
from jax.experimental.pallas import tpu as pltpu
from jax.experimental.pallas import tpu_sc as plsc

NUM_CORES = 2
NUM_SUBCORES = 16
NW = NUM_CORES * NUM_SUBCORES

BLKB = 4


@functools.partial(jax.jit, static_argnames=("b_per_w",))
def _embed(idx, table, b_per_w):
    bsz, hist = idx.shape
    d = table.shape[1]
    n_blk = b_per_w // BLKB
    assert b_per_w % BLKB == 0 and n_blk % 2 == 0 and n_blk >= 4
    assert hist % 8 == 0
    mesh = plsc.VectorSubcoreMesh(
        core_axis_name="c", subcore_axis_name="s",
        num_cores=NUM_CORES, num_subcores=NUM_SUBCORES)

    @functools.partial(
        pl.kernel,
        out_type=jax.ShapeDtypeStruct((bsz, hist, d), jnp.float32),
        mesh=mesh,
        scratch_types=[
            pltpu.VMEM((BLKB * hist,), jnp.int32),
            pltpu.VMEM((BLKB * hist,), jnp.int32),
            pltpu.VMEM((BLKB, hist, d), jnp.float32),
            pltpu.VMEM((BLKB, hist, d), jnp.float32),
            pltpu.SemaphoreType.DMA,
            pltpu.SemaphoreType.DMA,
            pltpu.SemaphoreType.DMA,
            pltpu.SemaphoreType.DMA,
        ],
        compiler_params=pltpu.CompilerParams(use_tc_tiling_on_sc=False),
    )
    def k(table_hbm, idx_hbm, out_hbm, idx0, idx1, rows0, rows1,
          sem_g0, sem_g1, sem_s0, sem_s1):
        wid = lax.axis_index("s") * NUM_CORES + lax.axis_index("c")
        base = wid * b_per_w

        def fire_gather(g, idx_v, rows_v, sem):
            boff = base + g * BLKB
            pltpu.sync_copy(
                idx_hbm.at[pl.ds(boff * hist, BLKB * hist)], idx_v)
            for j in range(BLKB):
                pltpu.async_copy(
                    table_hbm.at[idx_v.at[pl.ds(j * hist, hist)]],
                    rows_v.at[j], sem)

        def wait_gather(idx_v, rows_v, sem):
            for j in range(BLKB):
                pltpu.make_async_copy(
                    table_hbm.at[idx_v.at[pl.ds(j * hist, hist)]],
                    rows_v.at[j], sem).wait()

        def fire_store(g, rows_v, sem):
            pltpu.async_copy(
                rows_v, out_hbm.at[pl.ds(base + g * BLKB, BLKB)], sem)

        def wait_store(g, rows_v, sem):
            pltpu.make_async_copy(
                rows_v, out_hbm.at[pl.ds(base + g * BLKB, BLKB)], sem).wait()

        fire_gather(0, idx0, rows0, sem_g0)
        fire_gather(1, idx1, rows1, sem_g1)
        wait_gather(idx0, rows0, sem_g0)
        fire_store(0, rows0, sem_s0)

        def step(kk, _):
            ga = 2 * kk + 1
            gb = 2 * kk + 2
            wait_store(ga - 1, rows0, sem_s0)
            fire_gather(gb, idx0, rows0, sem_g0)
            wait_gather(idx1, rows1, sem_g1)
            fire_store(ga, rows1, sem_s1)
            wait_store(gb - 1, rows1, sem_s1)
            fire_gather(gb + 1, idx1, rows1, sem_g1)
            wait_gather(idx0, rows0, sem_g0)
            fire_store(gb, rows0, sem_s0)
            return ()

        lax.fori_loop(0, (n_blk - 2) // 2, step, (), unroll=False)

        gl = n_blk - 1
        wait_store(gl - 1, rows0, sem_s0)
        wait_gather(idx1, rows1, sem_g1)
        fire_store(gl, rows1, sem_s1)
        wait_store(gl, rows1, sem_s1)

    return k(table, idx.reshape(bsz * hist))


def kernel(agent_types, table):
    b, h = agent_types.shape
    idx = agent_types.astype(jnp.int32)
    return _embed(idx, table, b // NW)


# speedup vs baseline: 1.7204x; 1.7204x over previous
import functools

import jax
import jax.numpy as jnp
from jax import lax
from jax.experimental import pallas as pl
from jax.experimental.pallas import tpu as pltpu
from jax.experimental.pallas import tpu_sc as plsc

NUM_CORES = 2
NUM_SUBCORES = 16
NW = NUM_CORES * NUM_SUBCORES
LANES = 16

HBLK = 25


@jax.jit
def _embed(idx_t, table_t):
    hist, bsz = idx_t.shape
    d = 64
    vocab = table_t.shape[0] // d
    dt_n, di_n = d // 8, 8
    bt_n = bsz // 128
    bt_per_w = bt_n // NW
    n_hb = hist // HBLK
    assert hist % HBLK == 0 and bt_n % NW == 0

    mesh = plsc.VectorSubcoreMesh(
        core_axis_name="c", subcore_axis_name="s",
        num_cores=NUM_CORES, num_subcores=NUM_SUBCORES)

    @functools.partial(
        pl.kernel,
        out_type=jax.ShapeDtypeStruct((hist, dt_n, bt_n, 8, 128),
                                      jnp.float32),
        mesh=mesh,
        scratch_types=[
            pltpu.VMEM((d * vocab,), jnp.float32),
            pltpu.VMEM((HBLK, bt_per_w * 128), jnp.int32),
            pltpu.VMEM((dt_n, 8, 128), jnp.float32),
            pltpu.VMEM((dt_n, 8, 128), jnp.float32),
            pltpu.SemaphoreType.DMA,
            pltpu.SemaphoreType.DMA,
        ],
        compiler_params=pltpu.CompilerParams(
            use_tc_tiling_on_sc=False, needs_layout_passes=False),
    )
    def k(tab_hbm, idx_hbm, out_hbm, tab_v, idx_blk, stage0, stage1,
          sem0, sem1):
        wid = lax.axis_index("s") * NUM_CORES + lax.axis_index("c")
        col0 = wid * (bt_per_w * 128)
        pltpu.sync_copy(tab_hbm, tab_v)

        def fill(hl, btl, stage):
            regs = [idx_blk[hl, pl.ds(btl * 128 + g * LANES, LANES)]
                    for g in range(8)]

            def dt_body(dt, _):
                for di in range(di_n):
                    off = (dt * 8 + di) * vocab
                    for g in range(8):
                        stage[dt, di, pl.ds(g * LANES, LANES)] = (
                            plsc.load_gather(tab_v, [regs[g] + off]))
                return ()

            lax.fori_loop(0, dt_n, dt_body, (), unroll=False)

        def fire(h, btl, stage, sem):
            pltpu.async_copy(
                stage, out_hbm.at[h, pl.ds(0, dt_n), wid * bt_per_w + btl],
                sem)

        def wait(h, btl, stage, sem):
            pltpu.make_async_copy(
                stage, out_hbm.at[h, pl.ds(0, dt_n), wid * bt_per_w + btl],
                sem).wait()

        def hb_body(hb, _):
            pltpu.sync_copy(
                idx_hbm.at[pl.ds(hb * HBLK, HBLK),
                           pl.ds(col0, bt_per_w * 128)],
                idx_blk)

            def hl_body(hl, _):
                h = hb * HBLK + hl
                first = (hb == 0) & (hl == 0)

                @pl.when(jnp.logical_not(first))
                def _():
                    wait(h, 0, stage0, sem0)
                fill(hl, 0, stage0)
                fire(h, 0, stage0, sem0)

                @pl.when(jnp.logical_not(first))
                def _():
                    wait(h, 1, stage1, sem1)
                fill(hl, 1, stage1)
                fire(h, 1, stage1, sem1)

                wait(h, 2, stage0, sem0)
                fill(hl, 2, stage0)
                fire(h, 2, stage0, sem0)

                wait(h, 3, stage1, sem1)
                fill(hl, 3, stage1)
                fire(h, 3, stage1, sem1)
                return ()

            lax.fori_loop(0, HBLK, hl_body, (), unroll=False)
            return ()

        lax.fori_loop(0, n_hb, hb_body, (), unroll=False)
        wait(hist - 1, 2, stage0, sem0)
        wait(hist - 1, 3, stage1, sem1)

    return k(table_t, idx_t)


def kernel(agent_types, table):
    b, h = agent_types.shape
    d = table.shape[1]
    idx_t = agent_types.T.astype(jnp.int32)
    table_t = table.T.reshape(d * table.shape[0])
    out5 = _embed(idx_t, table_t)
    return out5.transpose(2, 4, 0, 1, 3).reshape(b, h, d)


# speedup vs baseline: 8.5958x; 4.9965x over previous
import functools

import jax
import jax.numpy as jnp
from jax import lax
from jax.experimental import pallas as pl
from jax.experimental.pallas import tpu as pltpu
from jax.experimental.pallas import tpu_sc as plsc

NUM_CORES = 2
NUM_SUBCORES = 16
NW = NUM_CORES * NUM_SUBCORES
LANES = 16

HBLK = 25


@jax.jit
def _embed(idx_t, table_t):
    hist, bsz = idx_t.shape
    d = 64
    vocab = table_t.shape[0] // d
    dt_n, di_n = d // 8, 8
    bt_n = bsz // 128
    bt_per_w = bt_n // NW
    n_hb = hist // HBLK
    assert hist % HBLK == 0 and bt_n % NW == 0

    mesh = plsc.VectorSubcoreMesh(
        core_axis_name="c", subcore_axis_name="s",
        num_cores=NUM_CORES, num_subcores=NUM_SUBCORES)

    @functools.partial(
        pl.kernel,
        out_type=jax.ShapeDtypeStruct((hist, dt_n, bt_n, 8, 128),
                                      jnp.float32),
        mesh=mesh,
        scratch_types=[
            pltpu.VMEM((d * vocab,), jnp.float32),
            pltpu.VMEM((HBLK, bt_per_w * 128), jnp.int32),
            pltpu.VMEM((dt_n, 8, 128), jnp.float32),
            pltpu.VMEM((dt_n, 8, 128), jnp.float32),
            pltpu.SemaphoreType.DMA,
            pltpu.SemaphoreType.DMA,
        ],
        compiler_params=pltpu.CompilerParams(
            use_tc_tiling_on_sc=False, needs_layout_passes=False),
    )
    def k(tab_hbm, idx_hbm, out_hbm, tab_v, idx_blk, stage0, stage1,
          sem0, sem1):
        wid = lax.axis_index("s") * NUM_CORES + lax.axis_index("c")
        col0 = wid * (bt_per_w * 128)
        pltpu.sync_copy(tab_hbm, tab_v)

        def fill(hl, btl, stage):
            regs = [idx_blk[hl, pl.ds(btl * 128 + g * LANES, LANES)]
                    for g in range(8)]

            @plsc.parallel_loop(0, dt_n * di_n, 1, unroll=2)
            def _(j):
                off = j * vocab
                vals = [plsc.load_gather(tab_v, [regs[g] + off])
                        for g in range(8)]
                for g in range(8):
                    stage[j // di_n, j % di_n, pl.ds(g * LANES, LANES)] = (
                        vals[g])

        def fire(h, btl, stage, sem):
            pltpu.async_copy(
                stage, out_hbm.at[h, pl.ds(0, dt_n), wid * bt_per_w + btl],
                sem)

        def wait(h, btl, stage, sem):
            pltpu.make_async_copy(
                stage, out_hbm.at[h, pl.ds(0, dt_n), wid * bt_per_w + btl],
                sem).wait()

        def hb_body(hb, _):
            pltpu.sync_copy(
                idx_hbm.at[pl.ds(hb * HBLK, HBLK),
                           pl.ds(col0, bt_per_w * 128)],
                idx_blk)

            def hl_body(hl, _):
                h = hb * HBLK + hl
                first = (hb == 0) & (hl == 0)

                @pl.when(jnp.logical_not(first))
                def _():
                    wait(h, 0, stage0, sem0)
                fill(hl, 0, stage0)
                fire(h, 0, stage0, sem0)

                @pl.when(jnp.logical_not(first))
                def _():
                    wait(h, 1, stage1, sem1)
                fill(hl, 1, stage1)
                fire(h, 1, stage1, sem1)

                wait(h, 2, stage0, sem0)
                fill(hl, 2, stage0)
                fire(h, 2, stage0, sem0)

                wait(h, 3, stage1, sem1)
                fill(hl, 3, stage1)
                fire(h, 3, stage1, sem1)
                return ()

            lax.fori_loop(0, HBLK, hl_body, (), unroll=False)
            return ()

        lax.fori_loop(0, n_hb, hb_body, (), unroll=False)
        wait(hist - 1, 2, stage0, sem0)
        wait(hist - 1, 3, stage1, sem1)

    return k(table_t, idx_t)


def kernel(agent_types, table):
    b, h = agent_types.shape
    d = table.shape[1]
    idx_t = agent_types.T.astype(jnp.int32)
    table_t = table.T.reshape(d * table.shape[0])
    out5 = _embed(idx_t, table_t)
    return out5.transpose(2, 4, 0, 1, 3).reshape(b, h, d)


# speedup vs baseline: 9.1774x; 1.0677x over previous
import functools

import jax
import jax.numpy as jnp
from jax import lax
from jax.experimental import pallas as pl
from jax.experimental.pallas import tpu as pltpu
from jax.experimental.pallas import tpu_sc as plsc

NUM_CORES = 2
NUM_SUBCORES = 16
NW = NUM_CORES * NUM_SUBCORES
LANES = 16


@jax.jit
def _embed(idx4, table_t):
    h8_n, bt_n, hi_n, bi_n = idx4.shape
    hist = h8_n * hi_n
    d = 64
    vocab = table_t.shape[0] // d
    dt_n = d // 8
    bt_per_w = bt_n // NW
    assert h8_n % 2 == 1 and h8_n >= 3 and bt_n % NW == 0

    mesh = plsc.VectorSubcoreMesh(
        core_axis_name="c", subcore_axis_name="s",
        num_cores=NUM_CORES, num_subcores=NUM_SUBCORES)

    @functools.partial(
        pl.kernel,
        out_type=jax.ShapeDtypeStruct((hist, dt_n, bt_n, 8, 128),
                                      jnp.float32),
        mesh=mesh,
        scratch_types=[
            pltpu.VMEM((d * vocab,), jnp.float32),
            pltpu.VMEM((bt_per_w, hi_n, bi_n), jnp.int32),
            pltpu.VMEM((bt_per_w, hi_n, bi_n), jnp.int32),
            pltpu.VMEM((dt_n, 8, 128), jnp.float32),
            pltpu.VMEM((dt_n, 8, 128), jnp.float32),
            pltpu.SemaphoreType.DMA,
            pltpu.SemaphoreType.DMA,
            pltpu.SemaphoreType.DMA,
            pltpu.SemaphoreType.DMA,
        ],
        compiler_params=pltpu.CompilerParams(
            use_tc_tiling_on_sc=False, needs_layout_passes=False),
    )
    def k(tab_hbm, idx_hbm, out_hbm, tab_v, blk_a, blk_b, stage0, stage1,
          sem_a, sem_b, sem0, sem1):
        wid = lax.axis_index("s") * NUM_CORES + lax.axis_index("c")
        bt0 = wid * bt_per_w
        pltpu.sync_copy(tab_hbm, tab_v)

        def fire_idx(h8, blk, sem):
            pltpu.async_copy(
                idx_hbm.at[h8, pl.ds(bt0, bt_per_w)], blk, sem)

        def wait_idx(h8, blk, sem):
            pltpu.make_async_copy(
                idx_hbm.at[h8, pl.ds(bt0, bt_per_w)], blk, sem).wait()

        def fill(blk, btl, hi, stage):
            regs = [blk[btl, hi, pl.ds(g * LANES, LANES)] for g in range(8)]

            @plsc.parallel_loop(0, dt_n * 8, 1, unroll=2)
            def _(j):
                off = j * vocab
                vals = [plsc.load_gather(tab_v, [regs[g] + off])
                        for g in range(8)]
                for g in range(8):
                    stage[j // 8, j % 8, pl.ds(g * LANES, LANES)] = vals[g]

        def fire_st(h, btl, stage, sem):
            pltpu.async_copy(
                stage, out_hbm.at[h, pl.ds(0, dt_n), bt0 + btl], sem)

        def wait_st(h, btl, stage, sem):
            pltpu.make_async_copy(
                stage, out_hbm.at[h, pl.ds(0, dt_n), bt0 + btl], sem).wait()

        def block(h8, blk, first):
            def hi_body(hi, _):
                h = h8 * hi_n + hi
                skip = first & (hi == 0)

                @pl.when(jnp.logical_not(skip))
                def _():
                    wait_st(h, 0, stage0, sem0)
                fill(blk, 0, hi, stage0)
                fire_st(h, 0, stage0, sem0)

                @pl.when(jnp.logical_not(skip))
                def _():
                    wait_st(h, 1, stage1, sem1)
                fill(blk, 1, hi, stage1)
                fire_st(h, 1, stage1, sem1)

                wait_st(h, 2, stage0, sem0)
                fill(blk, 2, hi, stage0)
                fire_st(h, 2, stage0, sem0)

                wait_st(h, 3, stage1, sem1)
                fill(blk, 3, hi, stage1)
                fire_st(h, 3, stage1, sem1)
                return ()

            lax.fori_loop(0, hi_n, hi_body, (), unroll=False)

        fire_idx(0, blk_a, sem_a)

        def pair(p, _):
            ha = 2 * p
            hb = 2 * p + 1
            wait_idx(ha, blk_a, sem_a)
            fire_idx(hb, blk_b, sem_b)
            block(ha, blk_a, p == 0)
            wait_idx(hb, blk_b, sem_b)
            fire_idx(hb + 1, blk_a, sem_a)
            block(hb, blk_b, False)
            return ()

        lax.fori_loop(0, (h8_n - 1) // 2, pair, (), unroll=False)

        wait_idx(h8_n - 1, blk_a, sem_a)
        block(h8_n - 1, blk_a, False)
        wait_st(hist - 1, 2, stage0, sem0)
        wait_st(hist - 1, 3, stage1, sem1)

    return k(table_t, idx4)


def kernel(agent_types, table):
    b, h = agent_types.shape
    d = table.shape[1]
    idx4 = (agent_types.astype(jnp.int32)
            .reshape(b // 128, 128, h // 8, 8)
            .transpose(2, 0, 3, 1))
    table_t = table.T.reshape(d * table.shape[0])
    out5 = _embed(idx4, table_t)
    return out5.transpose(2, 4, 0, 1, 3).reshape(b, h, d)
